# baseline (speedup 1.0000x reference)
import functools

import jax
import jax.numpy as jnp
from jax import lax
from jax.experimental import pallas as pl
from jax.experimental.pallas import tpu as pltpu
from jax.experimental.pallas import tpu_sc as plsc

N = 10000
N_PAD = 10240
D = 128
BLK = 1024
NBLK = N_PAD // BLK
EPS = 1e-5

NC = 2
NS = 16
NTILES = NC * NS
CPT = D // NTILES
ECH = 4000


def _elu(v):
    return jnp.where(v > 0, v, jnp.exp(v) - 1.0)


@functools.partial(jax.jit, static_argnames=("nchunks",))
def _spmm(xT, src, dst, w, *, nchunks):
    mesh = plsc.VectorSubcoreMesh(core_axis_name="c", subcore_axis_name="s")

    def body(xT_h, src_h, dst_h, w_h, out_h,
             tmp, xpk, acc, sb0, db0, wb0, sb1, db1, wb1, sem0, sem1):
        wid = lax.axis_index("s") * NC + lax.axis_index("c")
        row0 = wid * CPT
        for p in range(CPT // 2):
            for r in range(2):
                pltpu.sync_copy(xT_h.at[row0 + 2 * p + r, :],
                                tmp.at[pl.ds(r * N_PAD, N_PAD)])

            @pl.loop(0, N_PAD // 16, unroll=4)
            def _(i):
                a = tmp[pl.ds(i * 16, 16)]
                b = tmp[pl.ds(N_PAD + i * 16, 16)]
                pk = plsc.pack(a, b, format=plsc.PackFormat.INTERLEAVED)
                xpk[pl.ds(p * N_PAD + i * 16, 16)] = plsc.bitcast(pk, jnp.int32)

        zero16 = jnp.zeros((16,), jnp.float32)

        @pl.loop(0, CPT * N_PAD // 16, unroll=4)
        def _(i):
            acc[pl.ds(i * 16, 16)] = zero16

        bufs = ((sb0, db0, wb0, sem0), (sb1, db1, wb1, sem1))

        def issue(k, b):
            sb, db, wb, sem = bufs[b]
            pltpu.async_copy(src_h.at[pl.ds(k * ECH, ECH)], sb, sem)
            pltpu.async_copy(dst_h.at[pl.ds(k * ECH, ECH)], db, sem)
            pltpu.async_copy(w_h.at[pl.ds(k * ECH, ECH)], wb, sem)

        def wait(k, b):
            sb, db, wb, sem = bufs[b]
            pltpu.make_async_copy(src_h.at[pl.ds(k * ECH, ECH)], sb, sem).wait()
            pltpu.make_async_copy(dst_h.at[pl.ds(k * ECH, ECH)], db, sem).wait()
            pltpu.make_async_copy(w_h.at[pl.ds(k * ECH, ECH)], wb, sem).wait()

        issue(0, 0)
        issue(1, 1)

        @pl.loop(0, nchunks, step=2)
        def _(k0):
            for b in range(2):
                k = k0 + b
                wait(k, b)
                sb, db, wb, _sem = bufs[b]

                @plsc.parallel_loop(0, ECH // 16, unroll=8)
                def _(j):
                    base = j * 16
                    sv = sb[pl.ds(base, 16)]
                    dv = db[pl.ds(base, 16)]
                    wv = wb[pl.ds(base, 16)]
                    for p in range(CPT // 2):
                        xp = xpk.at[pl.ds(p * N_PAD, N_PAD)]
                        gp = plsc.load_gather(xp, [sv])
                        a, b = plsc.unpack(plsc.bitcast(gp, jnp.bfloat16),
                                           format=plsc.PackFormat.INTERLEAVED)
                        a0 = acc.at[pl.ds((2 * p) * N_PAD, N_PAD)]
                        a1 = acc.at[pl.ds((2 * p + 1) * N_PAD, N_PAD)]
                        plsc.addupdate_scatter(a0, [dv], a * wv)
                        plsc.addupdate_scatter(a1, [dv], b * wv)

                @pl.when(k + 2 < nchunks)
                def _():
                    issue(k + 2, b)

        for c in range(CPT):
            pltpu.sync_copy(acc.at[pl.ds(c * N_PAD, N_PAD)], out_h.at[row0 + c, :])

    spmm = pl.kernel(
        body,
        out_type=jax.ShapeDtypeStruct((D, N_PAD), jnp.float32),
        mesh=mesh,
        compiler_params=pltpu.CompilerParams(needs_layout_passes=False),
        scratch_types=[
            pltpu.VMEM((2 * N_PAD,), jnp.float32),
            pltpu.VMEM(((CPT // 2) * N_PAD,), jnp.int32),
            pltpu.VMEM((CPT * N_PAD,), jnp.float32),
            pltpu.VMEM((ECH,), jnp.int32),
            pltpu.VMEM((ECH,), jnp.int32),
            pltpu.VMEM((ECH,), jnp.float32),
            pltpu.VMEM((ECH,), jnp.int32),
            pltpu.VMEM((ECH,), jnp.int32),
            pltpu.VMEM((ECH,), jnp.float32),
            pltpu.SemaphoreType.DMA,
            pltpu.SemaphoreType.DMA,
        ],
    )
    return spmm(xT, src, dst, w)


def _mm_in(wT, hT, bcol):
    def body(w_ref, b_ref, h_ref, o_ref):
        i = pl.program_id(0)
        o = jnp.dot(w_ref[...], h_ref[...], preferred_element_type=jnp.float32)
        o = o + b_ref[:, 0:1]
        col = i * BLK + lax.broadcasted_iota(jnp.int32, (D, BLK), 1)
        o_ref[...] = jnp.where(col < N, o, 0.0)

    return pl.pallas_call(
        body,
        grid=(NBLK,),
        in_specs=[
            pl.BlockSpec((D, D), lambda i: (0, 0)),
            pl.BlockSpec((D, 128), lambda i: (0, 0)),
            pl.BlockSpec((D, BLK), lambda i: (0, i)),
        ],
        out_specs=pl.BlockSpec((D, BLK), lambda i: (0, i)),
        out_shape=jax.ShapeDtypeStruct((D, N_PAD), jnp.float32),
    )(wT, bcol, hT)


def _stats_contrib(ex, es):
    s1 = jnp.concatenate(
        [jnp.sum(ex, axis=1, keepdims=True), jnp.sum(es, axis=1, keepdims=True)], axis=0)
    q1 = jnp.concatenate(
        [jnp.sum(ex * ex, axis=1, keepdims=True), jnp.sum(es * es, axis=1, keepdims=True)],
        axis=0)
    lane = lax.broadcasted_iota(jnp.int32, (2 * D, 128), 1)
    return jnp.where(lane == 0, s1, 0.0) + jnp.where(lane == 1, q1, 0.0)


def _apply(gb, wT, bcol, xT, sT, resT):
    with_res = resT is not None

    def body(*refs):
        if with_res:
            gb_ref, w_ref, b_ref, x_ref, s_ref, r_ref, o_ref, st_scr = refs
        else:
            gb_ref, w_ref, b_ref, x_ref, s_ref, o_ref, st_scr = refs
        ph = pl.program_id(0)
        i = pl.program_id(1)

        @pl.when((ph == 0) & (i == 0))
        def _():
            st_scr[...] = jnp.zeros_like(st_scr)

        @pl.when(ph == 0)
        def _():
            st_scr[...] += _stats_contrib(_elu(x_ref[...]), _elu(s_ref[...]))

        @pl.when(ph == 1)
        def _():
            inv_n = 1.0 / N
            mean = st_scr[:, 0:1] * inv_n
            var = st_scr[:, 1:2] * inv_n - mean * mean
            scale = gb_ref[:, 0:1] * lax.rsqrt(var + EPS)
            shift = gb_ref[:, 1:2] - mean * scale
            nx = _elu(x_ref[...]) * scale[:D] + shift[:D]
            ns = _elu(s_ref[...]) * scale[D:] + shift[D:]
            o = (jnp.dot(w_ref[:, :D], nx, preferred_element_type=jnp.float32)
                 + jnp.dot(w_ref[:, D:], ns, preferred_element_type=jnp.float32))
            o = o + b_ref[:, 0:1]
            if with_res:
                o = o + refs[5][...]
            col = i * BLK + lax.broadcasted_iota(jnp.int32, (D, BLK), 1)
            o_ref[...] = jnp.where(col < N, o, 0.0)

    in_specs = [
        pl.BlockSpec((2 * D, 128), lambda p, i: (0, 0)),
        pl.BlockSpec((D, 2 * D), lambda p, i: (0, 0)),
        pl.BlockSpec((D, 128), lambda p, i: (0, 0)),
        pl.BlockSpec((D, BLK), lambda p, i: (0, i)),
        pl.BlockSpec((D, BLK), lambda p, i: (0, i)),
    ]
    args = [gb, wT, bcol, xT, sT]
    if with_res:
        in_specs.append(pl.BlockSpec((D, BLK), lambda p, i: (0, i)))
        args.append(resT)

    return pl.pallas_call(
        body,
        grid=(2, NBLK),
        in_specs=in_specs,
        out_specs=pl.BlockSpec((D, BLK), lambda p, i: (0, i * p)),
        out_shape=jax.ShapeDtypeStruct((D, N_PAD), jnp.float32),
        scratch_shapes=[pltpu.VMEM((2 * D, 128), jnp.float32)],
    )(*args)


def _final(gb1, w2T, b2col, wmuT, bmucol, xT, inT):
    def body(gb_ref, w2_ref, b2_ref, wm_ref, bm_ref, x_ref, in_ref, o_ref, st_scr):
        ph = pl.program_id(0)
        i = pl.program_id(1)

        @pl.when((ph == 0) & (i == 0))
        def _():
            st_scr[...] = jnp.zeros_like(st_scr)

        @pl.when(ph == 0)
        def _():
            ex = _elu(x_ref[...])
            s1 = jnp.sum(ex, axis=1, keepdims=True)
            q1 = jnp.sum(ex * ex, axis=1, keepdims=True)
            lane = lax.broadcasted_iota(jnp.int32, (D, 128), 1)
            st_scr[...] += jnp.where(lane == 0, s1, 0.0) + jnp.where(lane == 1, q1, 0.0)

        @pl.when(ph == 1)
        def _():
            inv_n = 1.0 / N
            mean = st_scr[:, 0:1] * inv_n
            var = st_scr[:, 1:2] * inv_n - mean * mean
            scale = gb_ref[:, 0:1] * lax.rsqrt(var + EPS)
            shift = gb_ref[:, 1:2] - mean * scale
            nx = _elu(x_ref[...]) * scale + shift
            z = jnp.dot(w2_ref[...], nx, preferred_element_type=jnp.float32) + b2_ref[:, 0:1]
            z = _elu(z)
            mu = jnp.dot(wm_ref[...], z, preferred_element_type=jnp.float32)
            o_ref[...] = mu + bm_ref[:, 0:1] + in_ref[...]

    return pl.pallas_call(
        body,
        grid=(2, NBLK),
        in_specs=[
            pl.BlockSpec((D, 128), lambda p, i: (0, 0)),
            pl.BlockSpec((D, D), lambda p, i: (0, 0)),
            pl.BlockSpec((D, 128), lambda p, i: (0, 0)),
            pl.BlockSpec((8, D), lambda p, i: (0, 0)),
            pl.BlockSpec((8, 128), lambda p, i: (0, 0)),
            pl.BlockSpec((D, BLK), lambda p, i: (0, i)),
            pl.BlockSpec((8, BLK), lambda p, i: (0, i)),
        ],
        out_specs=pl.BlockSpec((8, BLK), lambda p, i: (0, i * p)),
        out_shape=jax.ShapeDtypeStruct((8, N_PAD), jnp.float32),
        scratch_shapes=[pltpu.VMEM((D, 128), jnp.float32)],
    )(gb1, w2T, b2col, wmuT, bmucol, xT, inT)


def _col(v, rows=None):
    rows = v.shape[0] if rows is None else rows
    out = jnp.zeros((rows, 128), jnp.float32)
    return out.at[: v.shape[0], 0].set(v)


def _gbcol(g, bb):
    out = jnp.zeros((g.shape[0], 128), jnp.float32)
    return out.at[:, 0].set(g).at[:, 1].set(bb)


def kernel(inputs, noise, edge_index, edge_weight, mask, params):
    f32 = jnp.float32
    x0 = inputs[0].astype(f32)
    nz = noise[0].astype(f32)
    nz_dim = nz.shape[1]
    hcat = jnp.concatenate([x0, nz], axis=1)
    kin = hcat.shape[1]
    hcatT = jnp.zeros((D, N_PAD), f32).at[:kin, :N].set(hcat.T)
    wcat = jnp.concatenate(
        [params["conv_inputs"]["W"], params["conv_noise"]["W"]], axis=0)
    wcatT = jnp.zeros((D, D), f32).at[:, :kin].set(wcat.T)
    bsum = params["conv_inputs"]["b"] + params["conv_noise"]["b"]

    src = edge_index[1].astype(jnp.int32)
    dst = edge_index[0].astype(jnp.int32)
    w = edge_weight.astype(f32)
    perm = jnp.argsort(dst)
    src, dst, w = src[perm], dst[perm], w[perm]
    nchunks = src.shape[0] // ECH

    x = _mm_in(wcatT, hcatT, _col(bsum, D))

    for i in range(5):
        p = params["rn%d" % i]
        s = _spmm(x, src, dst, w, nchunks=nchunks)
        x1 = _apply(_gbcol(p["g0"], p["bb0"]), p["W0"].T, _col(p["b0"], D), x, s, None)
        s1 = _spmm(x1, src, dst, w, nchunks=nchunks)
        x = _apply(_gbcol(p["g1"], p["bb1"]), p["W1"].T, _col(p["b1"], D), x1, s1, x)

    wmuT = jnp.zeros((8, D), f32).at[:3, :].set(params["Wmu"].T)
    bmucol = jnp.zeros((8, 128), f32).at[:3, 0].set(params["bmu"])
    inT = jnp.zeros((8, N_PAD), f32).at[:3, :N].set(x0.T)
    muT = _final(_gbcol(params["bn2_g"], params["bn2_b"]), params["W2"].T,
                 _col(params["b2"], D), wmuT, bmucol, x, inT)

    mu = muT[:3, :N].T[None]
    y = jnp.broadcast_to(params["fc_logvar"], mu.shape)
    return (mu, y)


# speedup vs baseline: 5.8757x; 5.8757x over previous
import functools

import jax
import jax.numpy as jnp
from jax import lax
from jax.experimental import pallas as pl
from jax.experimental.pallas import tpu as pltpu
from jax.experimental.pallas import tpu_sc as plsc

N = 10000
N_PAD = 10240
D = 128
BLK = 1024
NBLK = N_PAD // BLK
EPS = 1e-5

NC = 2
NS = 16
NTILES = NC * NS
CPT = D // NTILES
ECH = 6400


def _elu(v):
    return jnp.where(v > 0, v, jnp.exp(v) - 1.0)


@functools.partial(jax.jit, static_argnames=("nchunks",))
def _spmm(xT, src, dst, w, *, nchunks):
    mesh = plsc.VectorSubcoreMesh(core_axis_name="c", subcore_axis_name="s")

    def body(xT_h, src_h, dst_h, w_h, out_h,
             tmp, xpk, acc, sb0, db0, wb0, sb1, db1, wb1, sem0, sem1):
        wid = lax.axis_index("s") * NC + lax.axis_index("c")
        row0 = wid * CPT
        for p in range(CPT // 2):
            for r in range(2):
                pltpu.sync_copy(xT_h.at[row0 + 2 * p + r, :],
                                tmp.at[pl.ds(r * N_PAD, N_PAD)])

            @pl.loop(0, N_PAD // 16, unroll=4)
            def _(i):
                a = tmp[pl.ds(i * 16, 16)]
                b = tmp[pl.ds(N_PAD + i * 16, 16)]
                pk = plsc.pack(a, b, format=plsc.PackFormat.INTERLEAVED)
                xpk[pl.ds(p * N_PAD + i * 16, 16)] = plsc.bitcast(pk, jnp.int32)

        zero16 = jnp.zeros((16,), jnp.float32)

        @pl.loop(0, CPT * N_PAD // 16, unroll=4)
        def _(i):
            acc[pl.ds(i * 16, 16)] = zero16

        bufs = ((sb0, db0, wb0, sem0), (sb1, db1, wb1, sem1))

        def issue(k, b):
            sb, db, wb, sem = bufs[b]
            pltpu.async_copy(src_h.at[pl.ds(k * ECH, ECH)], sb, sem)
            pltpu.async_copy(dst_h.at[pl.ds(k * ECH, ECH)], db, sem)
            pltpu.async_copy(w_h.at[pl.ds(k * ECH, ECH)], wb, sem)

        def wait(k, b):
            sb, db, wb, sem = bufs[b]
            pltpu.make_async_copy(src_h.at[pl.ds(k * ECH, ECH)], sb, sem).wait()
            pltpu.make_async_copy(dst_h.at[pl.ds(k * ECH, ECH)], db, sem).wait()
            pltpu.make_async_copy(w_h.at[pl.ds(k * ECH, ECH)], wb, sem).wait()

        issue(0, 0)
        issue(1, 1)

        @pl.loop(0, nchunks, step=2)
        def _(k0):
            for b in range(2):
                k = k0 + b
                wait(k, b)
                sb, db, wb, _sem = bufs[b]

                @plsc.parallel_loop(0, ECH // 16, unroll=8)
                def _(j):
                    base = j * 16
                    sv = sb[pl.ds(base, 16)]
                    dv = db[pl.ds(base, 16)]
                    wv = wb[pl.ds(base, 16)]
                    for p in range(CPT // 2):
                        xp = xpk.at[pl.ds(p * N_PAD, N_PAD)]
                        gp = plsc.load_gather(xp, [sv])
                        a, b = plsc.unpack(plsc.bitcast(gp, jnp.bfloat16),
                                           format=plsc.PackFormat.INTERLEAVED)
                        a0 = acc.at[pl.ds((2 * p) * N_PAD, N_PAD)]
                        a1 = acc.at[pl.ds((2 * p + 1) * N_PAD, N_PAD)]
                        plsc.addupdate_scatter(a0, [dv], a * wv)
                        plsc.addupdate_scatter(a1, [dv], b * wv)

                @pl.when(k + 2 < nchunks)
                def _():
                    issue(k + 2, b)

        for c in range(CPT):
            pltpu.sync_copy(acc.at[pl.ds(c * N_PAD, N_PAD)], out_h.at[row0 + c, :])

    spmm = pl.kernel(
        body,
        out_type=jax.ShapeDtypeStruct((D, N_PAD), jnp.float32),
        mesh=mesh,
        compiler_params=pltpu.CompilerParams(needs_layout_passes=False),
        scratch_types=[
            pltpu.VMEM((2 * N_PAD,), jnp.float32),
            pltpu.VMEM(((CPT // 2) * N_PAD,), jnp.int32),
            pltpu.VMEM((CPT * N_PAD,), jnp.float32),
            pltpu.VMEM((ECH,), jnp.int32),
            pltpu.VMEM((ECH,), jnp.int32),
            pltpu.VMEM((ECH,), jnp.float32),
            pltpu.VMEM((ECH,), jnp.int32),
            pltpu.VMEM((ECH,), jnp.int32),
            pltpu.VMEM((ECH,), jnp.float32),
            pltpu.SemaphoreType.DMA,
            pltpu.SemaphoreType.DMA,
        ],
    )
    return spmm(xT, src, dst, w)


def _mm_in(wT, hT, bcol):
    def body(w_ref, b_ref, h_ref, o_ref):
        i = pl.program_id(0)
        o = jnp.dot(w_ref[...], h_ref[...], preferred_element_type=jnp.float32)
        o = o + b_ref[:, 0:1]
        col = i * BLK + lax.broadcasted_iota(jnp.int32, (D, BLK), 1)
        o_ref[...] = jnp.where(col < N, o, 0.0)

    return pl.pallas_call(
        body,
        grid=(NBLK,),
        in_specs=[
            pl.BlockSpec((D, D), lambda i: (0, 0)),
            pl.BlockSpec((D, 128), lambda i: (0, 0)),
            pl.BlockSpec((D, BLK), lambda i: (0, i)),
        ],
        out_specs=pl.BlockSpec((D, BLK), lambda i: (0, i)),
        out_shape=jax.ShapeDtypeStruct((D, N_PAD), jnp.float32),
    )(wT, bcol, hT)


def _stats_contrib(ex, es):
    s1 = jnp.concatenate(
        [jnp.sum(ex, axis=1, keepdims=True), jnp.sum(es, axis=1, keepdims=True)], axis=0)
    q1 = jnp.concatenate(
        [jnp.sum(ex * ex, axis=1, keepdims=True), jnp.sum(es * es, axis=1, keepdims=True)],
        axis=0)
    lane = lax.broadcasted_iota(jnp.int32, (2 * D, 128), 1)
    return jnp.where(lane == 0, s1, 0.0) + jnp.where(lane == 1, q1, 0.0)


def _apply(gb, wT, bcol, xT, sT, resT):
    with_res = resT is not None

    def body(*refs):
        if with_res:
            gb_ref, w_ref, b_ref, x_ref, s_ref, r_ref, o_ref, st_scr = refs
        else:
            gb_ref, w_ref, b_ref, x_ref, s_ref, o_ref, st_scr = refs
        ph = pl.program_id(0)
        i = pl.program_id(1)

        @pl.when((ph == 0) & (i == 0))
        def _():
            st_scr[...] = jnp.zeros_like(st_scr)

        @pl.when(ph == 0)
        def _():
            st_scr[...] += _stats_contrib(_elu(x_ref[...]), _elu(s_ref[...]))

        @pl.when(ph == 1)
        def _():
            inv_n = 1.0 / N
            mean = st_scr[:, 0:1] * inv_n
            var = st_scr[:, 1:2] * inv_n - mean * mean
            scale = gb_ref[:, 0:1] * lax.rsqrt(var + EPS)
            shift = gb_ref[:, 1:2] - mean * scale
            nx = _elu(x_ref[...]) * scale[:D] + shift[:D]
            ns = _elu(s_ref[...]) * scale[D:] + shift[D:]
            o = (jnp.dot(w_ref[:, :D], nx, preferred_element_type=jnp.float32)
                 + jnp.dot(w_ref[:, D:], ns, preferred_element_type=jnp.float32))
            o = o + b_ref[:, 0:1]
            if with_res:
                o = o + refs[5][...]
            col = i * BLK + lax.broadcasted_iota(jnp.int32, (D, BLK), 1)
            o_ref[...] = jnp.where(col < N, o, 0.0)

    in_specs = [
        pl.BlockSpec((2 * D, 128), lambda p, i: (0, 0)),
        pl.BlockSpec((D, 2 * D), lambda p, i: (0, 0)),
        pl.BlockSpec((D, 128), lambda p, i: (0, 0)),
        pl.BlockSpec((D, BLK), lambda p, i: (0, i)),
        pl.BlockSpec((D, BLK), lambda p, i: (0, i)),
    ]
    args = [gb, wT, bcol, xT, sT]
    if with_res:
        in_specs.append(pl.BlockSpec((D, BLK), lambda p, i: (0, i)))
        args.append(resT)

    return pl.pallas_call(
        body,
        grid=(2, NBLK),
        in_specs=in_specs,
        out_specs=pl.BlockSpec((D, BLK), lambda p, i: (0, i * p)),
        out_shape=jax.ShapeDtypeStruct((D, N_PAD), jnp.float32),
        scratch_shapes=[pltpu.VMEM((2 * D, 128), jnp.float32)],
    )(*args)


def _final(gb1, w2T, b2col, wmuT, bmucol, xT, inT):
    def body(gb_ref, w2_ref, b2_ref, wm_ref, bm_ref, x_ref, in_ref, o_ref, st_scr):
        ph = pl.program_id(0)
        i = pl.program_id(1)

        @pl.when((ph == 0) & (i == 0))
        def _():
            st_scr[...] = jnp.zeros_like(st_scr)

        @pl.when(ph == 0)
        def _():
            ex = _elu(x_ref[...])
            s1 = jnp.sum(ex, axis=1, keepdims=True)
            q1 = jnp.sum(ex * ex, axis=1, keepdims=True)
            lane = lax.broadcasted_iota(jnp.int32, (D, 128), 1)
            st_scr[...] += jnp.where(lane == 0, s1, 0.0) + jnp.where(lane == 1, q1, 0.0)

        @pl.when(ph == 1)
        def _():
            inv_n = 1.0 / N
            mean = st_scr[:, 0:1] * inv_n
            var = st_scr[:, 1:2] * inv_n - mean * mean
            scale = gb_ref[:, 0:1] * lax.rsqrt(var + EPS)
            shift = gb_ref[:, 1:2] - mean * scale
            nx = _elu(x_ref[...]) * scale + shift
            z = jnp.dot(w2_ref[...], nx, preferred_element_type=jnp.float32) + b2_ref[:, 0:1]
            z = _elu(z)
            mu = jnp.dot(wm_ref[...], z, preferred_element_type=jnp.float32)
            o_ref[...] = mu + bm_ref[:, 0:1] + in_ref[...]

    return pl.pallas_call(
        body,
        grid=(2, NBLK),
        in_specs=[
            pl.BlockSpec((D, 128), lambda p, i: (0, 0)),
            pl.BlockSpec((D, D), lambda p, i: (0, 0)),
            pl.BlockSpec((D, 128), lambda p, i: (0, 0)),
            pl.BlockSpec((8, D), lambda p, i: (0, 0)),
            pl.BlockSpec((8, 128), lambda p, i: (0, 0)),
            pl.BlockSpec((D, BLK), lambda p, i: (0, i)),
            pl.BlockSpec((8, BLK), lambda p, i: (0, i)),
        ],
        out_specs=pl.BlockSpec((8, BLK), lambda p, i: (0, i * p)),
        out_shape=jax.ShapeDtypeStruct((8, N_PAD), jnp.float32),
        scratch_shapes=[pltpu.VMEM((D, 128), jnp.float32)],
    )(gb1, w2T, b2col, wmuT, bmucol, xT, inT)


def _col(v, rows=None):
    rows = v.shape[0] if rows is None else rows
    out = jnp.zeros((rows, 128), jnp.float32)
    return out.at[: v.shape[0], 0].set(v)


def _gbcol(g, bb):
    out = jnp.zeros((g.shape[0], 128), jnp.float32)
    return out.at[:, 0].set(g).at[:, 1].set(bb)


def kernel(inputs, noise, edge_index, edge_weight, mask, params):
    f32 = jnp.float32
    x0 = inputs[0].astype(f32)
    nz = noise[0].astype(f32)
    nz_dim = nz.shape[1]
    hcat = jnp.concatenate([x0, nz], axis=1)
    kin = hcat.shape[1]
    hcatT = jnp.zeros((D, N_PAD), f32).at[:kin, :N].set(hcat.T)
    wcat = jnp.concatenate(
        [params["conv_inputs"]["W"], params["conv_noise"]["W"]], axis=0)
    wcatT = jnp.zeros((D, D), f32).at[:, :kin].set(wcat.T)
    bsum = params["conv_inputs"]["b"] + params["conv_noise"]["b"]

    src = edge_index[1].astype(jnp.int32)
    dst = edge_index[0].astype(jnp.int32)
    w = edge_weight.astype(f32)
    nchunks = src.shape[0] // ECH

    x = _mm_in(wcatT, hcatT, _col(bsum, D))

    for i in range(5):
        p = params["rn%d" % i]
        s = _spmm(x, src, dst, w, nchunks=nchunks)
        x1 = _apply(_gbcol(p["g0"], p["bb0"]), p["W0"].T, _col(p["b0"], D), x, s, None)
        s1 = _spmm(x1, src, dst, w, nchunks=nchunks)
        x = _apply(_gbcol(p["g1"], p["bb1"]), p["W1"].T, _col(p["b1"], D), x1, s1, x)

    wmuT = jnp.zeros((8, D), f32).at[:3, :].set(params["Wmu"].T)
    bmucol = jnp.zeros((8, 128), f32).at[:3, 0].set(params["bmu"])
    inT = jnp.zeros((8, N_PAD), f32).at[:3, :N].set(x0.T)
    muT = _final(_gbcol(params["bn2_g"], params["bn2_b"]), params["W2"].T,
                 _col(params["b2"], D), wmuT, bmucol, x, inT)

    mu = muT[:3, :N].T[None]
    y = jnp.broadcast_to(params["fc_logvar"], mu.shape)
    return (mu, y)


# speedup vs baseline: 5.8786x; 1.0005x over previous
import functools

import jax
import jax.numpy as jnp
from jax import lax
from jax.experimental import pallas as pl
from jax.experimental.pallas import tpu as pltpu
from jax.experimental.pallas import tpu_sc as plsc

N = 10000
N_PAD = 10240
D = 128
BLK = 1024
NBLK = N_PAD // BLK
EPS = 1e-5

NC = 2
NS = 16
NTILES = NC * NS
CPT = D // NTILES
ECH = 6400


def _elu(v):
    return jnp.where(v > 0, v, jnp.exp(v) - 1.0)


@functools.partial(jax.jit, static_argnames=("nchunks",))
def _spmm(xT, src, dst, w, *, nchunks):
    mesh = plsc.VectorSubcoreMesh(core_axis_name="c", subcore_axis_name="s")

    def body(xT_h, src_h, dst_h, w_h, out_h,
             tmp, xpk, acc, sb0, db0, wb0, sb1, db1, wb1, sem0, sem1):
        wid = lax.axis_index("s") * NC + lax.axis_index("c")
        row0 = wid * CPT
        for p in range(CPT // 2):
            for r in range(2):
                pltpu.sync_copy(xT_h.at[row0 + 2 * p + r, :],
                                tmp.at[pl.ds(r * N_PAD, N_PAD)])

            @pl.loop(0, N_PAD // 16, unroll=4)
            def _(i):
                a = tmp[pl.ds(i * 16, 16)]
                b = tmp[pl.ds(N_PAD + i * 16, 16)]
                pk = plsc.pack(a, b, format=plsc.PackFormat.INTERLEAVED)
                xpk[pl.ds(p * N_PAD + i * 16, 16)] = plsc.bitcast(pk, jnp.int32)

        zero16 = jnp.zeros((16,), jnp.float32)

        @pl.loop(0, CPT * N_PAD // 16, unroll=4)
        def _(i):
            acc[pl.ds(i * 16, 16)] = zero16

        bufs = ((sb0, db0, wb0, sem0), (sb1, db1, wb1, sem1))

        def issue(k, b):
            sb, db, wb, sem = bufs[b]
            pltpu.async_copy(src_h.at[pl.ds(k * ECH, ECH)], sb, sem)
            pltpu.async_copy(dst_h.at[pl.ds(k * ECH, ECH)], db, sem)
            pltpu.async_copy(w_h.at[pl.ds(k * ECH, ECH)], wb, sem)

        def wait(k, b):
            sb, db, wb, sem = bufs[b]
            pltpu.make_async_copy(src_h.at[pl.ds(k * ECH, ECH)], sb, sem).wait()
            pltpu.make_async_copy(dst_h.at[pl.ds(k * ECH, ECH)], db, sem).wait()
            pltpu.make_async_copy(w_h.at[pl.ds(k * ECH, ECH)], wb, sem).wait()

        issue(0, 0)
        issue(1, 1)

        @pl.loop(0, nchunks, step=2)
        def _(k0):
            for b in range(2):
                k = k0 + b
                wait(k, b)
                sb, db, wb, _sem = bufs[b]

                @plsc.parallel_loop(0, ECH // 16, unroll=8)
                def _(j):
                    base = j * 16
                    sv = sb[pl.ds(base, 16)]
                    dv = db[pl.ds(base, 16)]
                    wv = wb[pl.ds(base, 16)]
                    for p in range(CPT // 2):
                        xp = xpk.at[pl.ds(p * N_PAD, N_PAD)]
                        gp = plsc.load_gather(xp, [sv])
                        g0, g1 = plsc.unpack(plsc.bitcast(gp, jnp.bfloat16),
                                             format=plsc.PackFormat.INTERLEAVED)
                        a0 = acc.at[pl.ds((2 * p) * N_PAD, N_PAD)]
                        a1 = acc.at[pl.ds((2 * p + 1) * N_PAD, N_PAD)]
                        plsc.addupdate_scatter(a0, [dv], g0 * wv)
                        plsc.addupdate_scatter(a1, [dv], g1 * wv)

                @pl.when(k + 2 < nchunks)
                def _():
                    issue(k + 2, b)

        for c in range(CPT):
            pltpu.sync_copy(acc.at[pl.ds(c * N_PAD, N_PAD)], out_h.at[row0 + c, :])

    spmm = pl.kernel(
        body,
        out_type=jax.ShapeDtypeStruct((D, N_PAD), jnp.float32),
        mesh=mesh,
        compiler_params=pltpu.CompilerParams(needs_layout_passes=False),
        scratch_types=[
            pltpu.VMEM((2 * N_PAD,), jnp.float32),
            pltpu.VMEM(((CPT // 2) * N_PAD,), jnp.int32),
            pltpu.VMEM((CPT * N_PAD,), jnp.float32),
            pltpu.VMEM((ECH,), jnp.int32),
            pltpu.VMEM((ECH,), jnp.int32),
            pltpu.VMEM((ECH,), jnp.float32),
            pltpu.VMEM((ECH,), jnp.int32),
            pltpu.VMEM((ECH,), jnp.int32),
            pltpu.VMEM((ECH,), jnp.float32),
            pltpu.SemaphoreType.DMA,
            pltpu.SemaphoreType.DMA,
        ],
    )
    return spmm(xT, src, dst, w)


def _mm_in(wT, hT, bcol):
    def body(w_ref, b_ref, h_ref, o_ref):
        i = pl.program_id(0)
        o = jnp.dot(w_ref[...], h_ref[...], preferred_element_type=jnp.float32)
        o = o + b_ref[:, 0:1]
        col = i * BLK + lax.broadcasted_iota(jnp.int32, (D, BLK), 1)
        o_ref[...] = jnp.where(col < N, o, 0.0)

    return pl.pallas_call(
        body,
        grid=(NBLK,),
        in_specs=[
            pl.BlockSpec((D, D), lambda i: (0, 0)),
            pl.BlockSpec((D, 128), lambda i: (0, 0)),
            pl.BlockSpec((D, BLK), lambda i: (0, i)),
        ],
        out_specs=pl.BlockSpec((D, BLK), lambda i: (0, i)),
        out_shape=jax.ShapeDtypeStruct((D, N_PAD), jnp.float32),
    )(wT, bcol, hT)


def _stats_contrib(ex, es):
    s1 = jnp.concatenate(
        [jnp.sum(ex, axis=1, keepdims=True), jnp.sum(es, axis=1, keepdims=True)], axis=0)
    q1 = jnp.concatenate(
        [jnp.sum(ex * ex, axis=1, keepdims=True), jnp.sum(es * es, axis=1, keepdims=True)],
        axis=0)
    lane = lax.broadcasted_iota(jnp.int32, (2 * D, 128), 1)
    return jnp.where(lane == 0, s1, 0.0) + jnp.where(lane == 1, q1, 0.0)


def _apply(gb, wT, bcol, xT, sT, resT):
    with_res = resT is not None

    def body(*refs):
        if with_res:
            gb_ref, w_ref, b_ref, x_ref, s_ref, r_ref, o_ref, st_scr = refs
        else:
            gb_ref, w_ref, b_ref, x_ref, s_ref, o_ref, st_scr = refs
        ph = pl.program_id(0)
        i = pl.program_id(1)

        @pl.when((ph == 0) & (i == 0))
        def _():
            st_scr[...] = jnp.zeros_like(st_scr)

        @pl.when(ph == 0)
        def _():
            st_scr[...] += _stats_contrib(_elu(x_ref[...]), _elu(s_ref[...]))

        @pl.when(ph == 1)
        def _():
            inv_n = 1.0 / N
            mean = st_scr[:, 0:1] * inv_n
            var = st_scr[:, 1:2] * inv_n - mean * mean
            scale = gb_ref[:, 0:1] * lax.rsqrt(var + EPS)
            shift = gb_ref[:, 1:2] - mean * scale
            nx = _elu(x_ref[...]) * scale[:D] + shift[:D]
            ns = _elu(s_ref[...]) * scale[D:] + shift[D:]
            o = (jnp.dot(w_ref[:, :D], nx, preferred_element_type=jnp.float32)
                 + jnp.dot(w_ref[:, D:], ns, preferred_element_type=jnp.float32))
            o = o + b_ref[:, 0:1]
            if with_res:
                o = o + refs[5][...]
            col = i * BLK + lax.broadcasted_iota(jnp.int32, (D, BLK), 1)
            o_ref[...] = jnp.where(col < N, o, 0.0)

    in_specs = [
        pl.BlockSpec((2 * D, 128), lambda p, i: (0, 0)),
        pl.BlockSpec((D, 2 * D), lambda p, i: (0, 0)),
        pl.BlockSpec((D, 128), lambda p, i: (0, 0)),
        pl.BlockSpec((D, BLK), lambda p, i: (0, i)),
        pl.BlockSpec((D, BLK), lambda p, i: (0, i)),
    ]
    args = [gb, wT, bcol, xT, sT]
    if with_res:
        in_specs.append(pl.BlockSpec((D, BLK), lambda p, i: (0, i)))
        args.append(resT)

    return pl.pallas_call(
        body,
        grid=(2, NBLK),
        in_specs=in_specs,
        out_specs=pl.BlockSpec((D, BLK), lambda p, i: (0, i * p)),
        out_shape=jax.ShapeDtypeStruct((D, N_PAD), jnp.float32),
        scratch_shapes=[pltpu.VMEM((2 * D, 128), jnp.float32)],
    )(*args)


def _final(gb1, w2T, b2col, wmuT, bmucol, xT, inT):
    def body(gb_ref, w2_ref, b2_ref, wm_ref, bm_ref, x_ref, in_ref, o_ref, st_scr):
        ph = pl.program_id(0)
        i = pl.program_id(1)

        @pl.when((ph == 0) & (i == 0))
        def _():
            st_scr[...] = jnp.zeros_like(st_scr)

        @pl.when(ph == 0)
        def _():
            ex = _elu(x_ref[...])
            s1 = jnp.sum(ex, axis=1, keepdims=True)
            q1 = jnp.sum(ex * ex, axis=1, keepdims=True)
            lane = lax.broadcasted_iota(jnp.int32, (D, 128), 1)
            st_scr[...] += jnp.where(lane == 0, s1, 0.0) + jnp.where(lane == 1, q1, 0.0)

        @pl.when(ph == 1)
        def _():
            inv_n = 1.0 / N
            mean = st_scr[:, 0:1] * inv_n
            var = st_scr[:, 1:2] * inv_n - mean * mean
            scale = gb_ref[:, 0:1] * lax.rsqrt(var + EPS)
            shift = gb_ref[:, 1:2] - mean * scale
            nx = _elu(x_ref[...]) * scale + shift
            z = jnp.dot(w2_ref[...], nx, preferred_element_type=jnp.float32) + b2_ref[:, 0:1]
            z = _elu(z)
            mu = jnp.dot(wm_ref[...], z, preferred_element_type=jnp.float32)
            o_ref[...] = mu + bm_ref[:, 0:1] + in_ref[...]

    return pl.pallas_call(
        body,
        grid=(2, NBLK),
        in_specs=[
            pl.BlockSpec((D, 128), lambda p, i: (0, 0)),
            pl.BlockSpec((D, D), lambda p, i: (0, 0)),
            pl.BlockSpec((D, 128), lambda p, i: (0, 0)),
            pl.BlockSpec((8, D), lambda p, i: (0, 0)),
            pl.BlockSpec((8, 128), lambda p, i: (0, 0)),
            pl.BlockSpec((D, BLK), lambda p, i: (0, i)),
            pl.BlockSpec((8, BLK), lambda p, i: (0, i)),
        ],
        out_specs=pl.BlockSpec((8, BLK), lambda p, i: (0, i * p)),
        out_shape=jax.ShapeDtypeStruct((8, N_PAD), jnp.float32),
        scratch_shapes=[pltpu.VMEM((D, 128), jnp.float32)],
    )(gb1, w2T, b2col, wmuT, bmucol, xT, inT)


def _col(v, rows=None):
    rows = v.shape[0] if rows is None else rows
    out = jnp.zeros((rows, 128), jnp.float32)
    return out.at[: v.shape[0], 0].set(v)


def _gbcol(g, bb):
    out = jnp.zeros((g.shape[0], 128), jnp.float32)
    return out.at[:, 0].set(g).at[:, 1].set(bb)


def kernel(inputs, noise, edge_index, edge_weight, mask, params):
    f32 = jnp.float32
    x0 = inputs[0].astype(f32)
    nz = noise[0].astype(f32)
    hcat = jnp.concatenate([x0, nz], axis=1)
    kin = hcat.shape[1]
    hcatT = jnp.zeros((D, N_PAD), f32).at[:kin, :N].set(hcat.T)
    wcat = jnp.concatenate(
        [params["conv_inputs"]["W"], params["conv_noise"]["W"]], axis=0)
    wcatT = jnp.zeros((D, D), f32).at[:, :kin].set(wcat.T)
    bsum = params["conv_inputs"]["b"] + params["conv_noise"]["b"]

    src = edge_index[1].astype(jnp.int32)
    dst = edge_index[0].astype(jnp.int32)
    w = edge_weight.astype(f32)
    nchunks = src.shape[0] // ECH

    x = _mm_in(wcatT, hcatT, _col(bsum, D))

    for i in range(5):
        p = params["rn%d" % i]
        s = _spmm(x, src, dst, w, nchunks=nchunks)
        x1 = _apply(_gbcol(p["g0"], p["bb0"]), p["W0"].T, _col(p["b0"], D), x, s, None)
        s1 = _spmm(x1, src, dst, w, nchunks=nchunks)
        x = _apply(_gbcol(p["g1"], p["bb1"]), p["W1"].T, _col(p["b1"], D), x1, s1, x)

    wmuT = jnp.zeros((8, D), f32).at[:3, :].set(params["Wmu"].T)
    bmucol = jnp.zeros((8, 128), f32).at[:3, 0].set(params["bmu"])
    inT = jnp.zeros((8, N_PAD), f32).at[:3, :N].set(x0.T)
    muT = _final(_gbcol(params["bn2_g"], params["bn2_b"]), params["W2"].T,
                 _col(params["b2"], D), wmuT, bmucol, x, inT)

    mu = muT[:3, :N].T[None]
    y = jnp.broadcast_to(params["fc_logvar"], mu.shape)
    return (mu, y)
